# baseline (device time: 423692 ns/iter reference)
import jax
import jax.numpy as jnp
from jax import lax
from jax.experimental import pallas as pl
from jax.experimental.pallas import tpu as pltpu

N_Z = 4


def kernel(x, W):
    t, _ = x.shape
    _, v_shard = W.shape
    vh = v_shard // 2

    logits = jnp.dot(x, W, preferred_element_type=jnp.float32)

    def body(l_ref, out_ref, ring_ref, stats_send_ref,
             stats_recv_ref, stats_send_sems, stats_recv_sems,
             ring_send_sems, ring_recv_sems, cross_send_sems,
             cross_recv_sems, out_sems):
        my_x = lax.axis_index("x")
        my_y = lax.axis_index("y")
        my_z = lax.axis_index("z")
        partner = (1 - my_x, my_y, my_z)
        right = (my_x, my_y, (my_z + 1) % N_Z)

        barrier_sem = pltpu.get_barrier_semaphore()
        for d in range(1, N_Z):
            pl.semaphore_signal(
                barrier_sem, inc=1,
                device_id=(my_x, my_y, (my_z + d) % N_Z),
                device_id_type=pl.DeviceIdType.MESH)
        pl.semaphore_signal(barrier_sem, inc=1, device_id=partner,
                            device_id_type=pl.DeviceIdType.MESH)
        pl.semaphore_wait(barrier_sem, N_Z)

        m_loc = jnp.max(l_ref[...], axis=1, keepdims=True)
        for half in range(2):
            sl = slice(half * vh, (half + 1) * vh)
            l_ref[:, sl] = jnp.exp(l_ref[:, sl] - m_loc)
        s_loc = jnp.sum(l_ref[...], axis=1, keepdims=True)
        stats_send_ref[0] = m_loc
        stats_send_ref[1] = s_loc

        stats_rdmas = []
        for d in range(1, N_Z):
            r = pltpu.make_async_remote_copy(
                src_ref=stats_send_ref,
                dst_ref=stats_recv_ref.at[d - 1],
                send_sem=stats_send_sems.at[d - 1],
                recv_sem=stats_recv_sems.at[d - 1],
                device_id=(my_x, my_y, (my_z + d) % N_Z),
                device_id_type=pl.DeviceIdType.MESH)
            r.start()
            stats_rdmas.append(r)
        for r in stats_rdmas:
            r.wait()

        m_g = m_loc
        for k in range(N_Z - 1):
            m_g = jnp.maximum(m_g, stats_recv_ref[k, 0])
        s_g = s_loc * jnp.exp(m_loc - m_g)
        for k in range(N_Z - 1):
            s_g = s_g + stats_recv_ref[k, 1] * jnp.exp(
                stats_recv_ref[k, 0] - m_g)
        scale = jnp.exp(m_loc - m_g) / s_g
        for half in range(2):
            sl = slice(half * vh, (half + 1) * vh)
            l_ref[:, sl] = l_ref[:, sl] * scale

            @pl.when(my_x == half)
            def _():
                ring_ref[0] = l_ref[:, sl]

        vq = vh // 2

        def fwd(hop, sub):
            ssl = slice(sub * vq, (sub + 1) * vq)
            return pltpu.make_async_remote_copy(
                src_ref=ring_ref.at[hop, :, ssl],
                dst_ref=ring_ref.at[hop + 1, :, ssl],
                send_sem=ring_send_sems.at[hop * 2 + sub],
                recv_sem=ring_recv_sems.at[hop * 2 + sub],
                device_id=right, device_id_type=pl.DeviceIdType.MESH)

        ring_rdmas = {}
        for s in range(2):
            ring_rdmas[(0, s)] = fwd(0, s)
            ring_rdmas[(0, s)].start()

        out_copies = [pltpu.make_async_copy(
            l_ref, out_ref.at[:, pl.ds(my_z * v_shard, v_shard)],
            out_sems.at[0])]
        out_copies[0].start()

        cross_rdmas = []
        cross_recvs = []
        for h in range(N_Z - 1):
            origin = (my_z - h - 1) % N_Z
            off = origin * v_shard
            for s in range(2):
                ssl = slice(s * vq, (s + 1) * vq)
                ring_rdmas[(h, s)].wait_recv()
                if h + 1 < N_Z - 1:
                    ring_rdmas[(h + 1, s)] = fwd(h + 1, s)
                    ring_rdmas[(h + 1, s)].start()
                cr = pltpu.make_async_remote_copy(
                    src_ref=ring_ref.at[h + 1, :, ssl],
                    dst_ref=out_ref.at[
                        :, pl.ds(off + my_x * vh + s * vq, vq)],
                    send_sem=cross_send_sems.at[h * 2 + s],
                    recv_sem=cross_recv_sems.at[h * 2 + s],
                    device_id=partner, device_id_type=pl.DeviceIdType.MESH)
                cr.start()
                cross_rdmas.append(cr)
                cross_recvs.append(pltpu.make_async_remote_copy(
                    src_ref=ring_ref.at[h + 1, :, ssl],
                    dst_ref=out_ref.at[
                        :, pl.ds(off + (1 - my_x) * vh + s * vq, vq)],
                    send_sem=cross_send_sems.at[h * 2 + s],
                    recv_sem=cross_recv_sems.at[h * 2 + s],
                    device_id=partner, device_id_type=pl.DeviceIdType.MESH))
            c = pltpu.make_async_copy(
                ring_ref.at[h + 1],
                out_ref.at[:, pl.ds(off + my_x * vh, vh)],
                out_sems.at[h + 1])
            c.start()
            out_copies.append(c)

        for r in ring_rdmas.values():
            r.wait_send()
        for cr in cross_rdmas:
            cr.wait_send()
        for c in out_copies:
            c.wait()
        for rc in cross_recvs:
            rc.wait_recv()

    out = pl.pallas_call(
        body,
        out_shape=jax.ShapeDtypeStruct((t, N_Z * v_shard), jnp.float32),
        in_specs=[pl.BlockSpec(memory_space=pltpu.MemorySpace.VMEM)],
        out_specs=pl.BlockSpec(memory_space=pl.ANY),
        scratch_shapes=[
            pltpu.VMEM((N_Z, t, vh), jnp.float32),
            pltpu.VMEM((2, t, 1), jnp.float32),
            pltpu.VMEM((N_Z - 1, 2, t, 1), jnp.float32),
            pltpu.SemaphoreType.DMA((N_Z - 1,)),
            pltpu.SemaphoreType.DMA((N_Z - 1,)),
            pltpu.SemaphoreType.DMA((2 * (N_Z - 1),)),
            pltpu.SemaphoreType.DMA((2 * (N_Z - 1),)),
            pltpu.SemaphoreType.DMA((2 * (N_Z - 1),)),
            pltpu.SemaphoreType.DMA((2 * (N_Z - 1),)),
            pltpu.SemaphoreType.DMA((N_Z,)),
        ],
        compiler_params=pltpu.CompilerParams(
            collective_id=0, vmem_limit_bytes=64 * 1024 * 1024),
    )(logits)

    return lax.optimization_barrier(out)


# device time: 401295 ns/iter; 1.0558x vs baseline; 1.0558x over previous
import jax
import jax.numpy as jnp
from jax import lax
from jax.experimental import pallas as pl
from jax.experimental.pallas import tpu as pltpu

N_Z = 4
SUB = 4


def kernel(x, W):
    t, _ = x.shape
    _, v_shard = W.shape
    vh = v_shard // 2

    logits = jnp.dot(x, W, preferred_element_type=jnp.float32)

    def body(l_ref, out_ref, ring_ref, stats_send_ref,
             stats_recv_ref, stats_send_sems, stats_recv_sems,
             ring_send_sems, ring_recv_sems, cross_send_sems,
             cross_recv_sems, out_sems):
        my_x = lax.axis_index("x")
        my_y = lax.axis_index("y")
        my_z = lax.axis_index("z")
        partner = (1 - my_x, my_y, my_z)
        right = (my_x, my_y, (my_z + 1) % N_Z)

        barrier_sem = pltpu.get_barrier_semaphore()
        for d in range(1, N_Z):
            pl.semaphore_signal(
                barrier_sem, inc=1,
                device_id=(my_x, my_y, (my_z + d) % N_Z),
                device_id_type=pl.DeviceIdType.MESH)
        pl.semaphore_signal(barrier_sem, inc=1, device_id=partner,
                            device_id_type=pl.DeviceIdType.MESH)
        pl.semaphore_wait(barrier_sem, N_Z)

        m_loc = jnp.max(l_ref[...], axis=1, keepdims=True)
        for half in range(2):
            sl = slice(half * vh, (half + 1) * vh)
            l_ref[:, sl] = jnp.exp(l_ref[:, sl] - m_loc)
        s_loc = jnp.sum(l_ref[...], axis=1, keepdims=True)
        stats_send_ref[0] = m_loc
        stats_send_ref[1] = s_loc

        stats_rdmas = []
        for d in range(1, N_Z):
            r = pltpu.make_async_remote_copy(
                src_ref=stats_send_ref,
                dst_ref=stats_recv_ref.at[d - 1],
                send_sem=stats_send_sems.at[d - 1],
                recv_sem=stats_recv_sems.at[d - 1],
                device_id=(my_x, my_y, (my_z + d) % N_Z),
                device_id_type=pl.DeviceIdType.MESH)
            r.start()
            stats_rdmas.append(r)
        for r in stats_rdmas:
            r.wait()

        m_g = m_loc
        for k in range(N_Z - 1):
            m_g = jnp.maximum(m_g, stats_recv_ref[k, 0])
        s_g = s_loc * jnp.exp(m_loc - m_g)
        for k in range(N_Z - 1):
            s_g = s_g + stats_recv_ref[k, 1] * jnp.exp(
                stats_recv_ref[k, 0] - m_g)
        scale = jnp.exp(m_loc - m_g) / s_g
        for half in range(2):
            sl = slice(half * vh, (half + 1) * vh)
            l_ref[:, sl] = l_ref[:, sl] * scale

            @pl.when(my_x == half)
            def _():
                ring_ref[0] = l_ref[:, sl]

        vq = vh // SUB

        def fwd(hop, sub):
            ssl = slice(sub * vq, (sub + 1) * vq)
            return pltpu.make_async_remote_copy(
                src_ref=ring_ref.at[hop, :, ssl],
                dst_ref=ring_ref.at[hop + 1, :, ssl],
                send_sem=ring_send_sems.at[hop * SUB + sub],
                recv_sem=ring_recv_sems.at[hop * SUB + sub],
                device_id=right, device_id_type=pl.DeviceIdType.MESH)

        ring_rdmas = {}
        for s in range(SUB):
            ring_rdmas[(0, s)] = fwd(0, s)
            ring_rdmas[(0, s)].start()

        out_copies = [pltpu.make_async_copy(
            l_ref, out_ref.at[:, pl.ds(my_z * v_shard, v_shard)],
            out_sems.at[0])]
        out_copies[0].start()

        cross_rdmas = []
        cross_recvs = []
        for h in range(N_Z - 1):
            origin = (my_z - h - 1) % N_Z
            off = origin * v_shard
            for s in range(SUB):
                ssl = slice(s * vq, (s + 1) * vq)
                ring_rdmas[(h, s)].wait_recv()
                if h + 1 < N_Z - 1:
                    ring_rdmas[(h + 1, s)] = fwd(h + 1, s)
                    ring_rdmas[(h + 1, s)].start()
                cr = pltpu.make_async_remote_copy(
                    src_ref=ring_ref.at[h + 1, :, ssl],
                    dst_ref=out_ref.at[
                        :, pl.ds(off + my_x * vh + s * vq, vq)],
                    send_sem=cross_send_sems.at[h * SUB + s],
                    recv_sem=cross_recv_sems.at[h * SUB + s],
                    device_id=partner, device_id_type=pl.DeviceIdType.MESH)
                cr.start()
                cross_rdmas.append(cr)
                cross_recvs.append(pltpu.make_async_remote_copy(
                    src_ref=ring_ref.at[h + 1, :, ssl],
                    dst_ref=out_ref.at[
                        :, pl.ds(off + (1 - my_x) * vh + s * vq, vq)],
                    send_sem=cross_send_sems.at[h * SUB + s],
                    recv_sem=cross_recv_sems.at[h * SUB + s],
                    device_id=partner, device_id_type=pl.DeviceIdType.MESH))
            c = pltpu.make_async_copy(
                ring_ref.at[h + 1],
                out_ref.at[:, pl.ds(off + my_x * vh, vh)],
                out_sems.at[h + 1])
            c.start()
            out_copies.append(c)

        for r in ring_rdmas.values():
            r.wait_send()
        for cr in cross_rdmas:
            cr.wait_send()
        for c in out_copies:
            c.wait()
        for rc in cross_recvs:
            rc.wait_recv()

    out = pl.pallas_call(
        body,
        out_shape=jax.ShapeDtypeStruct((t, N_Z * v_shard), jnp.float32),
        in_specs=[pl.BlockSpec(memory_space=pltpu.MemorySpace.VMEM)],
        out_specs=pl.BlockSpec(memory_space=pl.ANY),
        scratch_shapes=[
            pltpu.VMEM((N_Z, t, vh), jnp.float32),
            pltpu.VMEM((2, t, 1), jnp.float32),
            pltpu.VMEM((N_Z - 1, 2, t, 1), jnp.float32),
            pltpu.SemaphoreType.DMA((N_Z - 1,)),
            pltpu.SemaphoreType.DMA((N_Z - 1,)),
            pltpu.SemaphoreType.DMA((SUB * (N_Z - 1),)),
            pltpu.SemaphoreType.DMA((SUB * (N_Z - 1),)),
            pltpu.SemaphoreType.DMA((SUB * (N_Z - 1),)),
            pltpu.SemaphoreType.DMA((SUB * (N_Z - 1),)),
            pltpu.SemaphoreType.DMA((N_Z,)),
        ],
        compiler_params=pltpu.CompilerParams(
            collective_id=0, vmem_limit_bytes=64 * 1024 * 1024),
    )(logits)

    return out


# device time: 390506 ns/iter; 1.0850x vs baseline; 1.0276x over previous
import jax
import jax.numpy as jnp
from jax import lax
from jax.experimental import pallas as pl
from jax.experimental.pallas import tpu as pltpu

N_Z = 4
SUB = 8


def kernel(x, W):
    t, _ = x.shape
    _, v_shard = W.shape
    vh = v_shard // 2

    logits = jnp.dot(x, W, preferred_element_type=jnp.float32)

    def body(l_ref, out_ref, ring_ref, stats_send_ref,
             stats_recv_ref, stats_send_sems, stats_recv_sems,
             ring_send_sems, ring_recv_sems, cross_send_sems,
             cross_recv_sems, out_sems):
        my_x = lax.axis_index("x")
        my_y = lax.axis_index("y")
        my_z = lax.axis_index("z")
        partner = (1 - my_x, my_y, my_z)
        right = (my_x, my_y, (my_z + 1) % N_Z)

        barrier_sem = pltpu.get_barrier_semaphore()
        for d in range(1, N_Z):
            pl.semaphore_signal(
                barrier_sem, inc=1,
                device_id=(my_x, my_y, (my_z + d) % N_Z),
                device_id_type=pl.DeviceIdType.MESH)
        pl.semaphore_signal(barrier_sem, inc=1, device_id=partner,
                            device_id_type=pl.DeviceIdType.MESH)
        pl.semaphore_wait(barrier_sem, N_Z)

        m_loc = jnp.max(l_ref[...], axis=1, keepdims=True)
        for half in range(2):
            sl = slice(half * vh, (half + 1) * vh)
            l_ref[:, sl] = jnp.exp(l_ref[:, sl] - m_loc)
        s_loc = jnp.sum(l_ref[...], axis=1, keepdims=True)
        stats_send_ref[0] = m_loc
        stats_send_ref[1] = s_loc

        stats_rdmas = []
        for d in range(1, N_Z):
            r = pltpu.make_async_remote_copy(
                src_ref=stats_send_ref,
                dst_ref=stats_recv_ref.at[d - 1],
                send_sem=stats_send_sems.at[d - 1],
                recv_sem=stats_recv_sems.at[d - 1],
                device_id=(my_x, my_y, (my_z + d) % N_Z),
                device_id_type=pl.DeviceIdType.MESH)
            r.start()
            stats_rdmas.append(r)
        for r in stats_rdmas:
            r.wait()

        m_g = m_loc
        for k in range(N_Z - 1):
            m_g = jnp.maximum(m_g, stats_recv_ref[k, 0])
        s_g = s_loc * jnp.exp(m_loc - m_g)
        for k in range(N_Z - 1):
            s_g = s_g + stats_recv_ref[k, 1] * jnp.exp(
                stats_recv_ref[k, 0] - m_g)
        scale = jnp.exp(m_loc - m_g) / s_g
        for half in range(2):
            sl = slice(half * vh, (half + 1) * vh)
            l_ref[:, sl] = l_ref[:, sl] * scale

            @pl.when(my_x == half)
            def _():
                ring_ref[0] = l_ref[:, sl]

        vq = vh // SUB

        def fwd(hop, sub):
            ssl = slice(sub * vq, (sub + 1) * vq)
            return pltpu.make_async_remote_copy(
                src_ref=ring_ref.at[hop, :, ssl],
                dst_ref=ring_ref.at[hop + 1, :, ssl],
                send_sem=ring_send_sems.at[hop * SUB + sub],
                recv_sem=ring_recv_sems.at[hop * SUB + sub],
                device_id=right, device_id_type=pl.DeviceIdType.MESH)

        ring_rdmas = {}
        for s in range(SUB):
            ring_rdmas[(0, s)] = fwd(0, s)
            ring_rdmas[(0, s)].start()

        out_copies = [pltpu.make_async_copy(
            l_ref, out_ref.at[:, pl.ds(my_z * v_shard, v_shard)],
            out_sems.at[0])]
        out_copies[0].start()

        cross_rdmas = []
        cross_recvs = []
        for h in range(N_Z - 1):
            origin = (my_z - h - 1) % N_Z
            off = origin * v_shard
            for s in range(SUB):
                ssl = slice(s * vq, (s + 1) * vq)
                ring_rdmas[(h, s)].wait_recv()
                if h + 1 < N_Z - 1:
                    ring_rdmas[(h + 1, s)] = fwd(h + 1, s)
                    ring_rdmas[(h + 1, s)].start()
                cr = pltpu.make_async_remote_copy(
                    src_ref=ring_ref.at[h + 1, :, ssl],
                    dst_ref=out_ref.at[
                        :, pl.ds(off + my_x * vh + s * vq, vq)],
                    send_sem=cross_send_sems.at[h * SUB + s],
                    recv_sem=cross_recv_sems.at[h * SUB + s],
                    device_id=partner, device_id_type=pl.DeviceIdType.MESH)
                cr.start()
                cross_rdmas.append(cr)
                cross_recvs.append(pltpu.make_async_remote_copy(
                    src_ref=ring_ref.at[h + 1, :, ssl],
                    dst_ref=out_ref.at[
                        :, pl.ds(off + (1 - my_x) * vh + s * vq, vq)],
                    send_sem=cross_send_sems.at[h * SUB + s],
                    recv_sem=cross_recv_sems.at[h * SUB + s],
                    device_id=partner, device_id_type=pl.DeviceIdType.MESH))
            c = pltpu.make_async_copy(
                ring_ref.at[h + 1],
                out_ref.at[:, pl.ds(off + my_x * vh, vh)],
                out_sems.at[h + 1])
            c.start()
            out_copies.append(c)

        for r in ring_rdmas.values():
            r.wait_send()
        for cr in cross_rdmas:
            cr.wait_send()
        for c in out_copies:
            c.wait()
        for rc in cross_recvs:
            rc.wait_recv()

    out = pl.pallas_call(
        body,
        out_shape=jax.ShapeDtypeStruct((t, N_Z * v_shard), jnp.float32),
        in_specs=[pl.BlockSpec(memory_space=pltpu.MemorySpace.VMEM)],
        out_specs=pl.BlockSpec(memory_space=pl.ANY),
        scratch_shapes=[
            pltpu.VMEM((N_Z, t, vh), jnp.float32),
            pltpu.VMEM((2, t, 1), jnp.float32),
            pltpu.VMEM((N_Z - 1, 2, t, 1), jnp.float32),
            pltpu.SemaphoreType.DMA((N_Z - 1,)),
            pltpu.SemaphoreType.DMA((N_Z - 1,)),
            pltpu.SemaphoreType.DMA((SUB * (N_Z - 1),)),
            pltpu.SemaphoreType.DMA((SUB * (N_Z - 1),)),
            pltpu.SemaphoreType.DMA((SUB * (N_Z - 1),)),
            pltpu.SemaphoreType.DMA((SUB * (N_Z - 1),)),
            pltpu.SemaphoreType.DMA((N_Z,)),
        ],
        compiler_params=pltpu.CompilerParams(
            collective_id=0, vmem_limit_bytes=64 * 1024 * 1024),
    )(logits)

    return out


# device time: 385858 ns/iter; 1.0981x vs baseline; 1.0120x over previous
import jax
import jax.numpy as jnp
from jax import lax
from jax.experimental import pallas as pl
from jax.experimental.pallas import tpu as pltpu

N_Z = 4
SUB = 16


def kernel(x, W):
    t, _ = x.shape
    _, v_shard = W.shape
    vh = v_shard // 2

    logits = jnp.dot(x, W, preferred_element_type=jnp.float32)

    def body(l_ref, out_ref, ring_ref, stats_send_ref,
             stats_recv_ref, stats_send_sems, stats_recv_sems,
             ring_send_sems, ring_recv_sems, cross_send_sems,
             cross_recv_sems, out_sems):
        my_x = lax.axis_index("x")
        my_y = lax.axis_index("y")
        my_z = lax.axis_index("z")
        partner = (1 - my_x, my_y, my_z)
        right = (my_x, my_y, (my_z + 1) % N_Z)

        barrier_sem = pltpu.get_barrier_semaphore()
        for d in range(1, N_Z):
            pl.semaphore_signal(
                barrier_sem, inc=1,
                device_id=(my_x, my_y, (my_z + d) % N_Z),
                device_id_type=pl.DeviceIdType.MESH)
        pl.semaphore_signal(barrier_sem, inc=1, device_id=partner,
                            device_id_type=pl.DeviceIdType.MESH)
        pl.semaphore_wait(barrier_sem, N_Z)

        m_loc = jnp.max(l_ref[...], axis=1, keepdims=True)
        for half in range(2):
            sl = slice(half * vh, (half + 1) * vh)
            l_ref[:, sl] = jnp.exp(l_ref[:, sl] - m_loc)
        s_loc = jnp.sum(l_ref[...], axis=1, keepdims=True)
        stats_send_ref[0] = m_loc
        stats_send_ref[1] = s_loc

        stats_rdmas = []
        for d in range(1, N_Z):
            r = pltpu.make_async_remote_copy(
                src_ref=stats_send_ref,
                dst_ref=stats_recv_ref.at[d - 1],
                send_sem=stats_send_sems.at[d - 1],
                recv_sem=stats_recv_sems.at[d - 1],
                device_id=(my_x, my_y, (my_z + d) % N_Z),
                device_id_type=pl.DeviceIdType.MESH)
            r.start()
            stats_rdmas.append(r)
        for r in stats_rdmas:
            r.wait()

        m_g = m_loc
        for k in range(N_Z - 1):
            m_g = jnp.maximum(m_g, stats_recv_ref[k, 0])
        s_g = s_loc * jnp.exp(m_loc - m_g)
        for k in range(N_Z - 1):
            s_g = s_g + stats_recv_ref[k, 1] * jnp.exp(
                stats_recv_ref[k, 0] - m_g)
        scale = jnp.exp(m_loc - m_g) / s_g
        for half in range(2):
            sl = slice(half * vh, (half + 1) * vh)
            l_ref[:, sl] = l_ref[:, sl] * scale

            @pl.when(my_x == half)
            def _():
                ring_ref[0] = l_ref[:, sl]

        vq = vh // SUB

        def fwd(hop, sub):
            ssl = slice(sub * vq, (sub + 1) * vq)
            return pltpu.make_async_remote_copy(
                src_ref=ring_ref.at[hop, :, ssl],
                dst_ref=ring_ref.at[hop + 1, :, ssl],
                send_sem=ring_send_sems.at[hop * SUB + sub],
                recv_sem=ring_recv_sems.at[hop * SUB + sub],
                device_id=right, device_id_type=pl.DeviceIdType.MESH)

        ring_rdmas = {}
        for s in range(SUB):
            ring_rdmas[(0, s)] = fwd(0, s)
            ring_rdmas[(0, s)].start()

        out_copies = [pltpu.make_async_copy(
            l_ref, out_ref.at[:, pl.ds(my_z * v_shard, v_shard)],
            out_sems.at[0])]
        out_copies[0].start()

        cross_rdmas = []
        cross_recvs = []
        for h in range(N_Z - 1):
            origin = (my_z - h - 1) % N_Z
            off = origin * v_shard
            for s in range(SUB):
                ssl = slice(s * vq, (s + 1) * vq)
                ring_rdmas[(h, s)].wait_recv()
                if h + 1 < N_Z - 1:
                    ring_rdmas[(h + 1, s)] = fwd(h + 1, s)
                    ring_rdmas[(h + 1, s)].start()
                cr = pltpu.make_async_remote_copy(
                    src_ref=ring_ref.at[h + 1, :, ssl],
                    dst_ref=out_ref.at[
                        :, pl.ds(off + my_x * vh + s * vq, vq)],
                    send_sem=cross_send_sems.at[h * SUB + s],
                    recv_sem=cross_recv_sems.at[h * SUB + s],
                    device_id=partner, device_id_type=pl.DeviceIdType.MESH)
                cr.start()
                cross_rdmas.append(cr)
                cross_recvs.append(pltpu.make_async_remote_copy(
                    src_ref=ring_ref.at[h + 1, :, ssl],
                    dst_ref=out_ref.at[
                        :, pl.ds(off + (1 - my_x) * vh + s * vq, vq)],
                    send_sem=cross_send_sems.at[h * SUB + s],
                    recv_sem=cross_recv_sems.at[h * SUB + s],
                    device_id=partner, device_id_type=pl.DeviceIdType.MESH))
            c = pltpu.make_async_copy(
                ring_ref.at[h + 1],
                out_ref.at[:, pl.ds(off + my_x * vh, vh)],
                out_sems.at[h + 1])
            c.start()
            out_copies.append(c)

        for r in ring_rdmas.values():
            r.wait_send()
        for cr in cross_rdmas:
            cr.wait_send()
        for c in out_copies:
            c.wait()
        for rc in cross_recvs:
            rc.wait_recv()

    out = pl.pallas_call(
        body,
        out_shape=jax.ShapeDtypeStruct((t, N_Z * v_shard), jnp.float32),
        in_specs=[pl.BlockSpec(memory_space=pltpu.MemorySpace.VMEM)],
        out_specs=pl.BlockSpec(memory_space=pl.ANY),
        scratch_shapes=[
            pltpu.VMEM((N_Z, t, vh), jnp.float32),
            pltpu.VMEM((2, t, 1), jnp.float32),
            pltpu.VMEM((N_Z - 1, 2, t, 1), jnp.float32),
            pltpu.SemaphoreType.DMA((N_Z - 1,)),
            pltpu.SemaphoreType.DMA((N_Z - 1,)),
            pltpu.SemaphoreType.DMA((SUB * (N_Z - 1),)),
            pltpu.SemaphoreType.DMA((SUB * (N_Z - 1),)),
            pltpu.SemaphoreType.DMA((SUB * (N_Z - 1),)),
            pltpu.SemaphoreType.DMA((SUB * (N_Z - 1),)),
            pltpu.SemaphoreType.DMA((N_Z,)),
        ],
        compiler_params=pltpu.CompilerParams(
            collective_id=0, vmem_limit_bytes=64 * 1024 * 1024),
    )(logits)

    return out
